# Initial kernel scaffold; baseline (speedup 1.0000x reference)
import jax
import jax.numpy as jnp
from jax.experimental import pallas as pl


def kernel(x, w1, b1, w2, b2, fc1w, fc1b, fc2w, fc2b):
    raise NotImplementedError("write your pallas kernel here")


# speedup vs baseline: 19.0050x; 19.0050x over previous
import jax
import jax.numpy as jnp
from jax.experimental import pallas as pl
from jax.experimental.pallas import tpu as pltpu

_C1 = 10
_C2 = 20
_FCP = 128
_B = 128


def _lenet_kernel(x_ref, w1_ref, b1_ref, w2_ref, b2_ref,
                  fc1wt_ref, fc1bb_ref, fc2wt_ref, fc2bb_ref,
                  o_ref,
                  x_scr, rm_scr, p1_scr, rm2_scr, fcin_scr):
    f32 = jnp.float32
    B = _B

    x_scr[...] = jnp.transpose(x_ref[...]).reshape(28, 28, B)

    def c1_body(c, carry):
        acc = jnp.zeros((24, 24, B), f32)
        for dj in range(5):
            xj = x_scr[:, dj:dj + 24, :]
            for di in range(5):
                acc = acc + w1_ref[di * 5 + dj, c] * xj[di:di + 24]
        acc = acc + b1_ref[0, c]
        rm_scr[...] = jnp.max(acc.reshape(12, 2, 24, B), axis=1)
        p = jnp.maximum(rm_scr[:, 0:24:2, :], rm_scr[:, 1:24:2, :])
        p1_scr[c] = jnp.maximum(p, 0.0)
        return carry

    jax.lax.fori_loop(0, _C1, c1_body, 0)

    def c2_co_body(co, carry):
        def ci_body(ci, acc):
            for dj in range(5):
                pj = p1_scr[ci, :, dj:dj + 8, :]
                for di in range(5):
                    acc = acc + w2_ref[di * 5 + dj, ci, co] * pj[di:di + 8]
            return acc

        acc0 = jnp.zeros((8, 8, B), f32) + b2_ref[0, co]
        acc = jax.lax.fori_loop(0, _C1, ci_body, acc0)
        rm2_scr[...] = jnp.max(acc.reshape(4, 2, 8, B), axis=1)
        p = jnp.maximum(rm2_scr[:, 0:8:2, :], rm2_scr[:, 1:8:2, :])
        fcin_scr[co] = jnp.maximum(p, 0.0).reshape(16, B)
        return carry

    jax.lax.fori_loop(0, _C2, c2_co_body, 0)

    fcin = fcin_scr[...].reshape(320, B)
    h1 = jnp.dot(fc1wt_ref[...], fcin, preferred_element_type=f32)
    h1 = jnp.maximum(h1 + fc1bb_ref[...], 0.0)
    logits = jnp.dot(fc2wt_ref[...], h1, preferred_element_type=f32)
    logits = logits + fc2bb_ref[...]
    m = jnp.max(logits, axis=0, keepdims=True)
    lse = m + jnp.log(jnp.sum(jnp.exp(logits - m), axis=0, keepdims=True))
    o_ref[0] = (logits - lse)[0:16, :]


def kernel(x, w1, b1, w2, b2, fc1w, fc1b, fc2w, fc2b):
    f32 = jnp.float32
    n = x.shape[0]
    xr = x.astype(f32).reshape(n, 28 * 28)
    nb = (n + _B - 1) // _B
    npad = nb * _B
    if npad != n:
        xr = jnp.pad(xr, ((0, npad - n), (0, 0)))

    fc1wt = (fc1w.reshape(4, 4, _C2, _FCP)
             .transpose(2, 0, 1, 3).reshape(320, _FCP).T)
    fc2wt = fc2w.T
    fc1bb = jnp.broadcast_to(fc1b.reshape(_FCP, 1), (_FCP, _B))
    fc2bb = jnp.broadcast_to(fc2b.reshape(_FCP, 1), (_FCP, _B))

    out = pl.pallas_call(
        _lenet_kernel,
        out_shape=jax.ShapeDtypeStruct((nb, 16, _B), f32),
        grid=(nb,),
        in_specs=[
            pl.BlockSpec((_B, 784), lambda b: (b, 0)),
            pl.BlockSpec(memory_space=pltpu.SMEM),
            pl.BlockSpec(memory_space=pltpu.SMEM),
            pl.BlockSpec(memory_space=pltpu.SMEM),
            pl.BlockSpec(memory_space=pltpu.SMEM),
            pl.BlockSpec((_FCP, 320), lambda b: (0, 0)),
            pl.BlockSpec((_FCP, _B), lambda b: (0, 0)),
            pl.BlockSpec((_FCP, _FCP), lambda b: (0, 0)),
            pl.BlockSpec((_FCP, _B), lambda b: (0, 0)),
        ],
        out_specs=pl.BlockSpec((1, 16, _B), lambda b: (b, 0, 0)),
        scratch_shapes=[
            pltpu.VMEM((28, 28, _B), f32),
            pltpu.VMEM((12, 24, _B), f32),
            pltpu.VMEM((_C1, 12, 12, _B), f32),
            pltpu.VMEM((4, 8, _B), f32),
            pltpu.VMEM((_C2, 16, _B), f32),
        ],
        compiler_params=pltpu.CompilerParams(
            dimension_semantics=("parallel",),
        ),
    )(xr, w1, b1, w2, b2, fc1wt, fc1bb, fc2wt, fc2bb)

    res = out[:, :10, :].transpose(0, 2, 1).reshape(npad, 10)
    return res[:n]


# speedup vs baseline: 54.5886x; 2.8723x over previous
import jax
import jax.numpy as jnp
from jax.experimental import pallas as pl
from jax.experimental.pallas import tpu as pltpu

_C1 = 10
_C2 = 20
_FCP = 128
_B = 128


def _lenet_kernel(x_ref, w1b_ref, w2b_ref, fc1wt_ref, fc2wt_ref,
                  o_ref,
                  x_scr, ones_scr, pm_scr, p1z_scr, c2m_scr, fcin_scr):
    f32 = jnp.float32
    B = _B

    x_scr[:, 0:28, :] = jnp.transpose(x_ref[...]).reshape(28, 28, B)
    x_scr[:, 28:32, :] = jnp.zeros((28, 4, B), f32)
    ones_scr[...] = jnp.ones((8, B), f32)
    p1z_scr[:, :, 12:16, :] = jnp.zeros((_C1, 12, 4, B), f32)

    def c1_body(ip, carry):
        def row(i):
            rhs = jnp.concatenate(
                [x_scr[i + di] for di in range(5)] + [ones_scr[...]], axis=0)
            return jnp.dot(w1b_ref[...], rhs,
                           preferred_element_type=f32)

        m = jnp.maximum(row(2 * ip), row(2 * ip + 1))
        pm_scr[...] = m
        p = jnp.maximum(pm_scr[0:240:2, :], pm_scr[1:240:2, :])
        p1z_scr[:, ip, 0:12, :] = jnp.maximum(p, 0.0).reshape(_C1, 12, B)
        return carry

    jax.lax.fori_loop(0, 12, c1_body, 0)

    def c2_body(hp, carry):
        def row(i):
            rhs = jnp.concatenate(
                [p1z_scr[ci, i + di]
                 for ci in range(_C1) for di in range(5)]
                + [ones_scr[...]], axis=0)
            return jnp.dot(w2b_ref[...], rhs,
                           preferred_element_type=f32)

        m = jnp.maximum(row(2 * hp), row(2 * hp + 1))
        c2m_scr[...] = m
        p = jnp.maximum(c2m_scr[0:160:2, :], c2m_scr[1:160:2, :])
        fcin_scr[hp] = jnp.maximum(p, 0.0)
        return carry

    jax.lax.fori_loop(0, 4, c2_body, 0)

    fcin = jnp.concatenate(
        [fcin_scr[...].reshape(320, B), ones_scr[...]], axis=0)
    h1 = jnp.dot(fc1wt_ref[...], fcin, preferred_element_type=f32)
    h1 = jnp.maximum(h1, 0.0)
    h1e = jnp.concatenate([h1, ones_scr[...]], axis=0)
    logits = jnp.dot(fc2wt_ref[...], h1e, preferred_element_type=f32)
    m = jnp.max(logits, axis=0, keepdims=True)
    lse = m + jnp.log(jnp.sum(jnp.exp(logits - m), axis=0, keepdims=True))
    o_ref[0] = (logits - lse)[0:16, :]


def _band(npos, width, ntap):
    p = jnp.arange(width)[:, None, None]
    j = jnp.arange(npos)[None, :, None]
    k = jnp.arange(ntap)[None, None, :]
    return (p == j + k).astype(jnp.float32)


def kernel(x, w1, b1, w2, b2, fc1w, fc1b, fc2w, fc2b):
    f32 = jnp.float32
    n = x.shape[0]
    xr = x.astype(f32).reshape(n, 28 * 28)
    nb = (n + _B - 1) // _B
    npad = nb * _B
    if npad != n:
        xr = jnp.pad(xr, ((0, npad - n), (0, 0)))

    w1r = w1.reshape(5, 5, _C1)
    t1 = jnp.einsum('dkc,pjk->cjdp', w1r, _band(24, 32, 5)).reshape(240, 160)
    b1c = jnp.broadcast_to(b1.reshape(_C1, 1), (_C1, 24)).reshape(240, 1)
    w1big = jnp.concatenate([t1, b1c, jnp.zeros((240, 7), f32)], axis=1)

    w2r = w2.reshape(5, 5, _C1, _C2)
    t2 = jnp.einsum('dkic,pjk->cjidp', w2r, _band(8, 16, 5)).reshape(160, 800)
    b2c = jnp.broadcast_to(b2.reshape(_C2, 1), (_C2, 8)).reshape(160, 1)
    w2big = jnp.concatenate([t2, b2c, jnp.zeros((160, 7), f32)], axis=1)

    fc1p = (fc1w.reshape(4, 4, _C2, _FCP)
            .transpose(0, 2, 1, 3).reshape(320, _FCP))
    fc1wt = jnp.concatenate(
        [fc1p.T, fc1b.reshape(_FCP, 1), jnp.zeros((_FCP, 7), f32)], axis=1)
    fc2wt = jnp.concatenate(
        [fc2w.T, fc2b.reshape(_FCP, 1), jnp.zeros((_FCP, 7), f32)], axis=1)

    out = pl.pallas_call(
        _lenet_kernel,
        out_shape=jax.ShapeDtypeStruct((nb, 16, _B), f32),
        grid=(nb,),
        in_specs=[
            pl.BlockSpec((_B, 784), lambda b: (b, 0)),
            pl.BlockSpec((240, 168), lambda b: (0, 0)),
            pl.BlockSpec((160, 808), lambda b: (0, 0)),
            pl.BlockSpec((_FCP, 328), lambda b: (0, 0)),
            pl.BlockSpec((_FCP, 136), lambda b: (0, 0)),
        ],
        out_specs=pl.BlockSpec((1, 16, _B), lambda b: (b, 0, 0)),
        scratch_shapes=[
            pltpu.VMEM((28, 32, _B), f32),
            pltpu.VMEM((8, _B), f32),
            pltpu.VMEM((240, _B), f32),
            pltpu.VMEM((_C1, 12, 16, _B), f32),
            pltpu.VMEM((160, _B), f32),
            pltpu.VMEM((4, 80, _B), f32),
        ],
        compiler_params=pltpu.CompilerParams(
            dimension_semantics=("parallel",),
        ),
    )(xr, w1big, w2big, fc1wt, fc2wt)

    res = out[:, :10, :].transpose(0, 2, 1).reshape(npad, 10)
    return res[:n]


# speedup vs baseline: 56.0660x; 1.0271x over previous
import jax
import jax.numpy as jnp
from jax.experimental import pallas as pl
from jax.experimental.pallas import tpu as pltpu

_C1 = 10
_C2 = 20
_FCP = 128
_B = 128


def _lenet_kernel(x_ref, w1b_ref, w2b_ref, fc1wt_ref, fc2wt_ref,
                  o_ref,
                  x_scr, ones_scr, p1z_scr, fcin_scr):
    f32 = jnp.float32
    B = _B

    x_scr[:, 0:28, :] = jnp.transpose(x_ref[...]).reshape(28, 28, B)
    x_scr[:, 28:32, :] = jnp.zeros((28, 4, B), f32)
    ones_scr[...] = jnp.ones((8, B), f32)
    p1z_scr[:, :, 12:16, :] = jnp.zeros((_C1, 12, 4, B), f32)

    def c1_body(ip, carry):
        def row(i):
            rhs = jnp.concatenate(
                [x_scr[i + di] for di in range(5)] + [ones_scr[...]], axis=0)
            return jnp.dot(w1b_ref[...], rhs,
                           preferred_element_type=f32)

        m = jnp.maximum(row(2 * ip), row(2 * ip + 1))
        p = jnp.maximum(m[0:120], m[120:240])
        p1z_scr[:, ip, 0:12, :] = jnp.maximum(p, 0.0).reshape(_C1, 12, B)
        return carry

    jax.lax.fori_loop(0, 12, c1_body, 0)

    def c2_body(hp, carry):
        def row(i):
            rhs = jnp.concatenate(
                [p1z_scr[ci, i + di]
                 for ci in range(_C1) for di in range(5)]
                + [ones_scr[...]], axis=0)
            return jnp.dot(w2b_ref[...], rhs,
                           preferred_element_type=f32)

        m = jnp.maximum(row(2 * hp), row(2 * hp + 1))
        p = jnp.maximum(m[0:80], m[80:160])
        fcin_scr[hp] = jnp.maximum(p, 0.0)
        return carry

    jax.lax.fori_loop(0, 4, c2_body, 0)

    fcin = jnp.concatenate(
        [fcin_scr[...].reshape(320, B), ones_scr[...]], axis=0)
    h1 = jnp.dot(fc1wt_ref[...], fcin, preferred_element_type=f32)
    h1 = jnp.maximum(h1, 0.0)
    h1e = jnp.concatenate([h1, ones_scr[...]], axis=0)
    logits = jnp.dot(fc2wt_ref[...], h1e, preferred_element_type=f32)
    m = jnp.max(logits, axis=0, keepdims=True)
    lse = m + jnp.log(jnp.sum(jnp.exp(logits - m), axis=0, keepdims=True))
    o_ref[0] = (logits - lse)[0:16, :]


def _band(npos, width, ntap):
    p = jnp.arange(width)[:, None, None]
    j = jnp.arange(npos)[None, :, None]
    k = jnp.arange(ntap)[None, None, :]
    return (p == j + k).astype(jnp.float32)


def kernel(x, w1, b1, w2, b2, fc1w, fc1b, fc2w, fc2b):
    f32 = jnp.float32
    n = x.shape[0]
    xr = x.astype(f32).reshape(n, 28 * 28)
    nb = (n + _B - 1) // _B
    npad = nb * _B
    if npad != n:
        xr = jnp.pad(xr, ((0, npad - n), (0, 0)))

    w1r = w1.reshape(5, 5, _C1)
    t1 = (jnp.einsum('dkc,pjk->cjdp', w1r, _band(24, 32, 5))
          .reshape(_C1, 12, 2, 5, 32).transpose(2, 0, 1, 3, 4)
          .reshape(240, 160))
    b1c = jnp.broadcast_to(b1.reshape(1, _C1, 1), (2, _C1, 12)).reshape(240, 1)
    w1big = jnp.concatenate([t1, b1c, jnp.zeros((240, 7), f32)], axis=1)

    w2r = w2.reshape(5, 5, _C1, _C2)
    t2 = (jnp.einsum('dkic,pjk->cjidp', w2r, _band(8, 16, 5))
          .reshape(_C2, 4, 2, _C1, 5, 16).transpose(2, 0, 1, 3, 4, 5)
          .reshape(160, 800))
    b2c = jnp.broadcast_to(b2.reshape(1, _C2, 1), (2, _C2, 4)).reshape(160, 1)
    w2big = jnp.concatenate([t2, b2c, jnp.zeros((160, 7), f32)], axis=1)

    fc1p = (fc1w.reshape(4, 4, _C2, _FCP)
            .transpose(0, 2, 1, 3).reshape(320, _FCP))
    fc1wt = jnp.concatenate(
        [fc1p.T, fc1b.reshape(_FCP, 1), jnp.zeros((_FCP, 7), f32)], axis=1)
    fc2wt = jnp.concatenate(
        [fc2w.T, fc2b.reshape(_FCP, 1), jnp.zeros((_FCP, 7), f32)], axis=1)

    out = pl.pallas_call(
        _lenet_kernel,
        out_shape=jax.ShapeDtypeStruct((nb, 16, _B), f32),
        grid=(nb,),
        in_specs=[
            pl.BlockSpec((_B, 784), lambda b: (b, 0)),
            pl.BlockSpec((240, 168), lambda b: (0, 0)),
            pl.BlockSpec((160, 808), lambda b: (0, 0)),
            pl.BlockSpec((_FCP, 328), lambda b: (0, 0)),
            pl.BlockSpec((_FCP, 136), lambda b: (0, 0)),
        ],
        out_specs=pl.BlockSpec((1, 16, _B), lambda b: (b, 0, 0)),
        scratch_shapes=[
            pltpu.VMEM((28, 32, _B), f32),
            pltpu.VMEM((8, _B), f32),
            pltpu.VMEM((_C1, 12, 16, _B), f32),
            pltpu.VMEM((4, 80, _B), f32),
        ],
        compiler_params=pltpu.CompilerParams(
            dimension_semantics=("parallel",),
        ),
    )(xr, w1big, w2big, fc1wt, fc2wt)

    res = out[:, :10, :].transpose(0, 2, 1).reshape(npad, 10)
    return res[:n]


# speedup vs baseline: 85.0808x; 1.5175x over previous
import jax
import jax.numpy as jnp
from jax.experimental import pallas as pl
from jax.experimental.pallas import tpu as pltpu

_C1 = 10
_C2 = 20
_FCP = 128
_B = 128


def _lenet_kernel(x_ref, w1b_ref, w2b_ref, fc1wt_ref, fc2wt_ref,
                  o_ref,
                  x_scr, ones_scr, p1z_scr, fcin_scr):
    f32 = jnp.float32
    B = _B

    x_scr[:, 0:28, :] = jnp.transpose(x_ref[...]).reshape(28, 28, B)
    x_scr[:, 28:32, :] = jnp.zeros((28, 4, B), f32)
    ones_scr[...] = jnp.ones((8, B), f32)
    p1z_scr[:, :, 12:16, :] = jnp.zeros((_C1, 12, 4, B), f32)

    for ip in range(12):
        rhs = jnp.concatenate(
            [x_scr[2 * ip + d] for d in range(6)] + [ones_scr[...]],
            axis=0)
        m = jnp.dot(w1b_ref[...], rhs,
                    preferred_element_type=f32)
        p = jnp.maximum(jnp.maximum(m[0:120], m[120:240]),
                        jnp.maximum(m[240:360], m[360:480]))
        p1z_scr[:, ip, 0:12, :] = jnp.maximum(p, 0.0).reshape(_C1, 12, B)

    for hp in range(4):
        rhs = jnp.concatenate(
            [p1z_scr[ci, 2 * hp + d]
             for ci in range(_C1) for d in range(6)]
            + [ones_scr[...]], axis=0)
        r = jnp.dot(w2b_ref[...], rhs,
                    preferred_element_type=f32)
        p = jnp.maximum(jnp.maximum(r[0:80], r[80:160]),
                        jnp.maximum(r[160:240], r[240:320]))
        fcin_scr[hp] = jnp.maximum(p, 0.0)

    fcin = jnp.concatenate(
        [fcin_scr[...].reshape(320, B), ones_scr[...]], axis=0)
    h1 = jnp.dot(fc1wt_ref[...], fcin, preferred_element_type=f32)
    h1 = jnp.maximum(h1, 0.0)
    h1e = jnp.concatenate([h1, ones_scr[...]], axis=0)
    logits = jnp.dot(fc2wt_ref[...], h1e, preferred_element_type=f32)
    m = jnp.max(logits, axis=0, keepdims=True)
    lse = m + jnp.log(jnp.sum(jnp.exp(logits - m), axis=0, keepdims=True))
    o_ref[0] = (logits - lse)[0:16, :]


def _band(npos, width, ntap):
    p = jnp.arange(width)[:, None, None]
    j = jnp.arange(npos)[None, :, None]
    k = jnp.arange(ntap)[None, None, :]
    return (p == j + k).astype(jnp.float32)


def kernel(x, w1, b1, w2, b2, fc1w, fc1b, fc2w, fc2b):
    f32 = jnp.float32
    n = x.shape[0]
    xr = x.astype(f32).reshape(n, 28 * 28)
    nb = (n + _B - 1) // _B
    npad = nb * _B
    if npad != n:
        xr = jnp.pad(xr, ((0, npad - n), (0, 0)))

    w1r = w1.reshape(5, 5, _C1)
    t1 = (jnp.einsum('dkc,pjk->cjdp', w1r, _band(24, 32, 5))
          .reshape(_C1, 12, 2, 5, 32).transpose(2, 0, 1, 3, 4)
          .reshape(240, 5, 32))
    w1c = jnp.zeros((2, 240, 6, 32), f32)
    w1c = w1c.at[0, :, 0:5, :].set(t1).at[1, :, 1:6, :].set(t1)
    b1c = jnp.broadcast_to(b1.reshape(1, 1, _C1, 1),
                           (2, 2, _C1, 12)).reshape(480, 1)
    w1big = jnp.concatenate(
        [w1c.reshape(480, 192), b1c, jnp.zeros((480, 7), f32)], axis=1)

    w2r = w2.reshape(5, 5, _C1, _C2)
    t2 = (jnp.einsum('dkic,pjk->cjidp', w2r, _band(8, 16, 5))
          .reshape(_C2, 4, 2, _C1, 5, 16).transpose(2, 0, 1, 3, 4, 5)
          .reshape(160, _C1, 5, 16))
    w2c = jnp.zeros((2, 160, _C1, 6, 16), f32)
    w2c = w2c.at[0, :, :, 0:5, :].set(t2).at[1, :, :, 1:6, :].set(t2)
    b2c = jnp.broadcast_to(b2.reshape(1, 1, _C2, 1),
                           (2, 2, _C2, 4)).reshape(320, 1)
    w2big = jnp.concatenate(
        [w2c.reshape(320, 960), b2c, jnp.zeros((320, 7), f32)], axis=1)

    fc1p = (fc1w.reshape(4, 4, _C2, _FCP)
            .transpose(0, 2, 1, 3).reshape(320, _FCP))
    fc1wt = jnp.concatenate(
        [fc1p.T, fc1b.reshape(_FCP, 1), jnp.zeros((_FCP, 7), f32)], axis=1)
    fc2wt = jnp.concatenate(
        [fc2w.T, fc2b.reshape(_FCP, 1), jnp.zeros((_FCP, 7), f32)], axis=1)

    out = pl.pallas_call(
        _lenet_kernel,
        out_shape=jax.ShapeDtypeStruct((nb, 16, _B), f32),
        grid=(nb,),
        in_specs=[
            pl.BlockSpec((_B, 784), lambda b: (b, 0)),
            pl.BlockSpec((480, 200), lambda b: (0, 0)),
            pl.BlockSpec((320, 968), lambda b: (0, 0)),
            pl.BlockSpec((_FCP, 328), lambda b: (0, 0)),
            pl.BlockSpec((_FCP, 136), lambda b: (0, 0)),
        ],
        out_specs=pl.BlockSpec((1, 16, _B), lambda b: (b, 0, 0)),
        scratch_shapes=[
            pltpu.VMEM((28, 32, _B), f32),
            pltpu.VMEM((8, _B), f32),
            pltpu.VMEM((_C1, 12, 16, _B), f32),
            pltpu.VMEM((4, 80, _B), f32),
        ],
        compiler_params=pltpu.CompilerParams(
            dimension_semantics=("parallel",),
        ),
    )(xr, w1big, w2big, fc1wt, fc2wt)

    res = out[:, :10, :].transpose(0, 2, 1).reshape(npad, 10)
    return res[:n]


# speedup vs baseline: 102.0163x; 1.1991x over previous
import jax
import jax.numpy as jnp
from jax.experimental import pallas as pl
from jax.experimental.pallas import tpu as pltpu

_C1 = 10
_C2 = 20
_FCP = 128
_B = 128


def _lenet_kernel(x_ref, w1b_ref, w2b_ref, fc1wt_ref, fc2wt_ref,
                  o_ref,
                  x_scr, ones_scr, p1z_scr, fcin_scr):
    f32 = jnp.float32
    B = _B

    for i in range(28):
        x_scr[i, 0:28, :] = jnp.transpose(x_ref[:, i, :])
    x_scr[:, 28:32, :] = jnp.zeros((28, 4, B), f32)
    ones_scr[...] = jnp.ones((8, B), f32)
    p1z_scr[:, :, 12:16, :] = jnp.zeros((_C1, 12, 4, B), f32)

    for ip in range(12):
        rhs = jnp.concatenate(
            [x_scr[2 * ip + d] for d in range(6)] + [ones_scr[...]],
            axis=0)
        m = jnp.dot(w1b_ref[...], rhs,
                    preferred_element_type=f32)
        p = jnp.maximum(jnp.maximum(m[0:120], m[120:240]),
                        jnp.maximum(m[240:360], m[360:480]))
        p1z_scr[:, ip, 0:12, :] = jnp.maximum(p, 0.0).reshape(_C1, 12, B)

    for hp in range(4):
        rhs = jnp.concatenate(
            [p1z_scr[ci, 2 * hp + d]
             for ci in range(_C1) for d in range(6)]
            + [ones_scr[...]], axis=0)
        r = jnp.dot(w2b_ref[...], rhs,
                    preferred_element_type=f32)
        p = jnp.maximum(jnp.maximum(r[0:80], r[80:160]),
                        jnp.maximum(r[160:240], r[240:320]))
        fcin_scr[hp] = jnp.maximum(p, 0.0)

    fcin = jnp.concatenate(
        [fcin_scr[...].reshape(320, B), ones_scr[...]], axis=0)
    h1 = jnp.dot(fc1wt_ref[...], fcin, preferred_element_type=f32)
    h1 = jnp.maximum(h1, 0.0)
    h1e = jnp.concatenate([h1, ones_scr[...]], axis=0)
    logits = jnp.dot(fc2wt_ref[...], h1e, preferred_element_type=f32)
    m = jnp.max(logits, axis=0, keepdims=True)
    lse = m + jnp.log(jnp.sum(jnp.exp(logits - m), axis=0, keepdims=True))
    o_ref[0] = jnp.transpose((logits - lse)[0:16, :])


def _band(npos, width, ntap):
    p = jnp.arange(width)[:, None, None]
    j = jnp.arange(npos)[None, :, None]
    k = jnp.arange(ntap)[None, None, :]
    return (p == j + k).astype(jnp.float32)


def kernel(x, w1, b1, w2, b2, fc1w, fc1b, fc2w, fc2b):
    f32 = jnp.float32
    n = x.shape[0]
    xr = x.astype(f32).reshape(n, 28, 28)
    nb = (n + _B - 1) // _B
    npad = nb * _B
    if npad != n:
        xr = jnp.pad(xr, ((0, npad - n), (0, 0), (0, 0)))

    w1r = w1.reshape(5, 5, _C1)
    t1 = (jnp.einsum('dkc,pjk->cjdp', w1r, _band(24, 32, 5))
          .reshape(_C1, 12, 2, 5, 32).transpose(2, 0, 1, 3, 4)
          .reshape(240, 5, 32))
    w1c = jnp.zeros((2, 240, 6, 32), f32)
    w1c = w1c.at[0, :, 0:5, :].set(t1).at[1, :, 1:6, :].set(t1)
    b1c = jnp.broadcast_to(b1.reshape(1, 1, _C1, 1),
                           (2, 2, _C1, 12)).reshape(480, 1)
    w1big = jnp.concatenate(
        [w1c.reshape(480, 192), b1c, jnp.zeros((480, 7), f32)], axis=1)

    w2r = w2.reshape(5, 5, _C1, _C2)
    t2 = (jnp.einsum('dkic,pjk->cjidp', w2r, _band(8, 16, 5))
          .reshape(_C2, 4, 2, _C1, 5, 16).transpose(2, 0, 1, 3, 4, 5)
          .reshape(160, _C1, 5, 16))
    w2c = jnp.zeros((2, 160, _C1, 6, 16), f32)
    w2c = w2c.at[0, :, :, 0:5, :].set(t2).at[1, :, :, 1:6, :].set(t2)
    b2c = jnp.broadcast_to(b2.reshape(1, 1, _C2, 1),
                           (2, 2, _C2, 4)).reshape(320, 1)
    w2big = jnp.concatenate(
        [w2c.reshape(320, 960), b2c, jnp.zeros((320, 7), f32)], axis=1)

    fc1p = (fc1w.reshape(4, 4, _C2, _FCP)
            .transpose(0, 2, 1, 3).reshape(320, _FCP))
    fc1wt = jnp.concatenate(
        [fc1p.T, fc1b.reshape(_FCP, 1), jnp.zeros((_FCP, 7), f32)], axis=1)
    fc2wt = jnp.concatenate(
        [fc2w.T, fc2b.reshape(_FCP, 1), jnp.zeros((_FCP, 7), f32)], axis=1)

    out = pl.pallas_call(
        _lenet_kernel,
        out_shape=jax.ShapeDtypeStruct((nb, _B, 16), f32),
        grid=(nb,),
        in_specs=[
            pl.BlockSpec((_B, 28, 28), lambda b: (b, 0, 0)),
            pl.BlockSpec((480, 200), lambda b: (0, 0)),
            pl.BlockSpec((320, 968), lambda b: (0, 0)),
            pl.BlockSpec((_FCP, 328), lambda b: (0, 0)),
            pl.BlockSpec((_FCP, 136), lambda b: (0, 0)),
        ],
        out_specs=pl.BlockSpec((1, _B, 16), lambda b: (b, 0, 0)),
        scratch_shapes=[
            pltpu.VMEM((28, 32, _B), f32),
            pltpu.VMEM((8, _B), f32),
            pltpu.VMEM((_C1, 12, 16, _B), f32),
            pltpu.VMEM((4, 80, _B), f32),
        ],
        compiler_params=pltpu.CompilerParams(
            dimension_semantics=("parallel",),
        ),
    )(xr, w1big, w2big, fc1wt, fc2wt)

    return out.reshape(npad, 16)[:n, :10]


# speedup vs baseline: 108.6204x; 1.0647x over previous
import jax
import jax.numpy as jnp
from jax.experimental import pallas as pl
from jax.experimental.pallas import tpu as pltpu

_C1 = 10
_C2 = 20
_FCP = 128
_B = 128


def _lenet_kernel(x_ref, w1b_ref, w2b_ref, fc1wt_ref, fc2wt_ref,
                  o_ref,
                  x_scr, ones_scr, p1z_scr, fcin_scr):
    f32 = jnp.float32
    bf16 = jnp.bfloat16
    B = _B

    for i in range(28):
        x_scr[i, 0:28, :] = jnp.transpose(x_ref[:, i, :]).astype(bf16)
    x_scr[:, 28:32, :] = jnp.zeros((28, 4, B), bf16)
    ones_scr[...] = jnp.ones((8, B), bf16)
    p1z_scr[:, :, 12:16, :] = jnp.zeros((_C1, 12, 4, B), bf16)

    for ip in range(12):
        rhs = jnp.concatenate(
            [x_scr[2 * ip + d] for d in range(6)] + [ones_scr[...]],
            axis=0)
        m = jnp.dot(w1b_ref[...], rhs,
                    preferred_element_type=f32)
        p = jnp.maximum(jnp.maximum(m[0:120], m[120:240]),
                        jnp.maximum(m[240:360], m[360:480]))
        p1z_scr[:, ip, 0:12, :] = (jnp.maximum(p, 0.0)
                                   .astype(bf16).reshape(_C1, 12, B))

    for hp in range(4):
        rhs = jnp.concatenate(
            [p1z_scr[ci, 2 * hp + d]
             for ci in range(_C1) for d in range(6)]
            + [ones_scr[...]], axis=0)
        r = jnp.dot(w2b_ref[...], rhs,
                    preferred_element_type=f32)
        p = jnp.maximum(jnp.maximum(r[0:80], r[80:160]),
                        jnp.maximum(r[160:240], r[240:320]))
        fcin_scr[hp] = jnp.maximum(p, 0.0)

    ones_f32 = jnp.ones((8, B), f32)
    fcin = jnp.concatenate(
        [fcin_scr[...].reshape(320, B), ones_f32], axis=0)
    h1 = jnp.dot(fc1wt_ref[...], fcin, preferred_element_type=f32)
    h1 = jnp.maximum(h1, 0.0)
    h1e = jnp.concatenate([h1, ones_f32], axis=0)
    logits = jnp.dot(fc2wt_ref[...], h1e, preferred_element_type=f32)
    m = jnp.max(logits, axis=0, keepdims=True)
    lse = m + jnp.log(jnp.sum(jnp.exp(logits - m), axis=0, keepdims=True))
    o_ref[0] = jnp.transpose((logits - lse)[0:16, :])


def _band(npos, width, ntap):
    p = jnp.arange(width)[:, None, None]
    j = jnp.arange(npos)[None, :, None]
    k = jnp.arange(ntap)[None, None, :]
    return (p == j + k).astype(jnp.float32)


def kernel(x, w1, b1, w2, b2, fc1w, fc1b, fc2w, fc2b):
    f32 = jnp.float32
    n = x.shape[0]
    xr = x.astype(f32).reshape(n, 28, 28)
    nb = (n + _B - 1) // _B
    npad = nb * _B
    if npad != n:
        xr = jnp.pad(xr, ((0, npad - n), (0, 0), (0, 0)))

    w1r = w1.reshape(5, 5, _C1)
    t1 = (jnp.einsum('dkc,pjk->cjdp', w1r, _band(24, 32, 5))
          .reshape(_C1, 12, 2, 5, 32).transpose(2, 0, 1, 3, 4)
          .reshape(240, 5, 32))
    w1c = jnp.zeros((2, 240, 6, 32), f32)
    w1c = w1c.at[0, :, 0:5, :].set(t1).at[1, :, 1:6, :].set(t1)
    b1c = jnp.broadcast_to(b1.reshape(1, 1, _C1, 1),
                           (2, 2, _C1, 12)).reshape(480, 1)
    w1big = jnp.concatenate(
        [w1c.reshape(480, 192), b1c,
         jnp.zeros((480, 7), f32)], axis=1).astype(jnp.bfloat16)

    w2r = w2.reshape(5, 5, _C1, _C2)
    t2 = (jnp.einsum('dkic,pjk->cjidp', w2r, _band(8, 16, 5))
          .reshape(_C2, 4, 2, _C1, 5, 16).transpose(2, 0, 1, 3, 4, 5)
          .reshape(160, _C1, 5, 16))
    w2c = jnp.zeros((2, 160, _C1, 6, 16), f32)
    w2c = w2c.at[0, :, :, 0:5, :].set(t2).at[1, :, :, 1:6, :].set(t2)
    b2c = jnp.broadcast_to(b2.reshape(1, 1, _C2, 1),
                           (2, 2, _C2, 4)).reshape(320, 1)
    w2big = jnp.concatenate(
        [w2c.reshape(320, 960), b2c,
         jnp.zeros((320, 7), f32)], axis=1).astype(jnp.bfloat16)

    fc1p = (fc1w.reshape(4, 4, _C2, _FCP)
            .transpose(0, 2, 1, 3).reshape(320, _FCP))
    fc1wt = jnp.concatenate(
        [fc1p.T, fc1b.reshape(_FCP, 1), jnp.zeros((_FCP, 7), f32)], axis=1)
    fc2wt = jnp.concatenate(
        [fc2w.T, fc2b.reshape(_FCP, 1), jnp.zeros((_FCP, 7), f32)], axis=1)

    out = pl.pallas_call(
        _lenet_kernel,
        out_shape=jax.ShapeDtypeStruct((nb, _B, 16), f32),
        grid=(nb,),
        in_specs=[
            pl.BlockSpec((_B, 28, 28), lambda b: (b, 0, 0)),
            pl.BlockSpec((480, 200), lambda b: (0, 0)),
            pl.BlockSpec((320, 968), lambda b: (0, 0)),
            pl.BlockSpec((_FCP, 328), lambda b: (0, 0)),
            pl.BlockSpec((_FCP, 136), lambda b: (0, 0)),
        ],
        out_specs=pl.BlockSpec((1, _B, 16), lambda b: (b, 0, 0)),
        scratch_shapes=[
            pltpu.VMEM((28, 32, _B), jnp.bfloat16),
            pltpu.VMEM((8, _B), jnp.bfloat16),
            pltpu.VMEM((_C1, 12, 16, _B), jnp.bfloat16),
            pltpu.VMEM((4, 80, _B), f32),
        ],
        compiler_params=pltpu.CompilerParams(
            dimension_semantics=("parallel",),
        ),
    )(xr, w1big, w2big, fc1wt, fc2wt)

    return out.reshape(npad, 16)[:n, :10]
